# baseline (speedup 1.0000x reference)
import functools

import jax
import jax.numpy as jnp
from jax import lax
from jax.experimental import pallas as pl
from jax.experimental.pallas import tpu as pltpu
from jax.experimental.pallas import tpu_sc as plsc

N_NODES = 10000
N_EDGES = 320000
D = 128

NC = 2
NS = 16
NW = NC * NS
CH = 128
TOTCH = -(-N_EDGES // (NS * CH)) * NS
EPAD = TOTCH * CH

F0 = 18
F1 = TOTCH // NS - F0
FMAX = max(F0, F1)

NPAD = 10112
RPT = NPAD // NS
NDEG = 10240
DPT = NDEG // NS

_mesh = plsc.VectorSubcoreMesh(core_axis_name="c", subcore_axis_name="s")


@functools.partial(
    pl.kernel,
    out_type=jax.ShapeDtypeStruct((NC, NDEG), jnp.float32),
    mesh=_mesh,
    scratch_types=[
        pltpu.VMEM_SHARED((NDEG,), jnp.float32),
        pltpu.VMEM((FMAX, CH), jnp.int32),
        pltpu.VMEM((CH,), jnp.float32),
    ],
)
def _deg_kernel(dst0_hbm, dst1_hbm, ones_hbm, zeros_hbm, degp_hbm,
                deg_sh, dst_v, ones_v):
    cid = lax.axis_index("c")
    sid = lax.axis_index("s")

    pltpu.sync_copy(zeros_hbm.at[pl.ds(sid * DPT, DPT)],
                    deg_sh.at[pl.ds(sid * DPT, DPT)])
    pltpu.sync_copy(ones_hbm, ones_v)

    @pl.when(cid == 0)
    def _():
        pltpu.sync_copy(dst0_hbm.at[sid], dst_v.at[pl.ds(0, F0)])

    @pl.when(cid == 1)
    def _():
        pltpu.sync_copy(dst1_hbm.at[sid], dst_v.at[pl.ds(0, F1)])

    plsc.subcore_barrier()
    nch = jnp.where(cid == 0, F0, F1)

    def body(j, _):
        pltpu.sync_copy(ones_v, deg_sh.at[dst_v.at[j]], add=True)
        return 0

    lax.fori_loop(0, nch, body, 0)
    plsc.subcore_barrier()

    pltpu.sync_copy(deg_sh.at[pl.ds(sid * DPT, DPT)],
                    degp_hbm.at[cid, pl.ds(sid * DPT, DPT)])


@functools.partial(
    pl.kernel,
    out_type=jax.ShapeDtypeStruct((NC, NPAD, D), jnp.float32),
    mesh=_mesh,
    scratch_types=[
        pltpu.VMEM_SHARED((NPAD, D), jnp.float32),
        pltpu.VMEM((2, CH), jnp.int32),
        pltpu.VMEM((2, CH), jnp.int32),
        pltpu.VMEM((2, CH, D), jnp.float32),
        pltpu.SemaphoreType.DMA,
        pltpu.SemaphoreType.DMA,
        pltpu.SemaphoreType.DMA,
        pltpu.SemaphoreType.DMA,
        pltpu.SemaphoreType.DMA,
        pltpu.SemaphoreType.DMA,
    ],
)
def _scatter_kernel(hs_hbm, src0_hbm, src1_hbm, dst0_hbm, dst1_hbm, p_hbm,
                    agg_sh, sidx_v, didx_v, rows_v,
                    gsem0, gsem1, isem0, isem1, dsem0, dsem1):
    cid = lax.axis_index("c")
    sid = lax.axis_index("s")

    z16 = jnp.zeros((16,), jnp.float32)

    def zbody(r, _):
        for c in range(D // 16):
            rows_v[0, r, pl.ds(c * 16, 16)] = z16
        return 0

    lax.fori_loop(0, CH, zbody, 0)
    for k in range(4):
        pltpu.sync_copy(rows_v.at[0],
                        agg_sh.at[pl.ds(sid * RPT + k * CH, CH)])
    pltpu.sync_copy(rows_v.at[0, pl.ds(0, RPT - 4 * CH)],
                    agg_sh.at[pl.ds(sid * RPT + 4 * CH, RPT - 4 * CH)])

    plsc.subcore_barrier()

    def run(src_hbm, dst_hbm, nch):
        pltpu.sync_copy(src_hbm.at[sid, 0], sidx_v.at[0])
        pltpu.sync_copy(dst_hbm.at[sid, 0], didx_v.at[0])
        pltpu.async_copy(hs_hbm.at[sidx_v.at[0]], rows_v.at[0], gsem0)
        pltpu.async_copy(src_hbm.at[sid, 1], sidx_v.at[1], isem1)
        pltpu.async_copy(dst_hbm.at[sid, 1], didx_v.at[1], dsem1)

        def body(j, _):
            buf = lax.rem(j, 2)

            def halfstep(b, nb, gsem_b, gsem_nb, isem_b, isem_nb,
                         dsem_b, dsem_nb):
                @pl.when(j + 1 < nch)
                def _():
                    pltpu.make_async_copy(src_hbm.at[sid, j + 1], sidx_v.at[nb],
                                          isem_nb).wait()
                    pltpu.async_copy(hs_hbm.at[sidx_v.at[nb]], rows_v.at[nb],
                                     gsem_nb)

                pltpu.make_async_copy(hs_hbm.at[sidx_v.at[b]], rows_v.at[b],
                                      gsem_b).wait()

                @pl.when(j + 2 < nch)
                def _():
                    pltpu.async_copy(src_hbm.at[sid, j + 2], sidx_v.at[b],
                                     isem_b)

                @pl.when(j >= 1)
                def _():
                    pltpu.make_async_copy(dst_hbm.at[sid, j], didx_v.at[b],
                                          dsem_b).wait()

                pltpu.sync_copy(rows_v.at[b], agg_sh.at[didx_v.at[b]], add=True)

                @pl.when(j + 2 < nch)
                def _():
                    pltpu.async_copy(dst_hbm.at[sid, j + 2], didx_v.at[b],
                                     dsem_b)

            @pl.when(buf == 0)
            def _():
                halfstep(0, 1, gsem0, gsem1, isem0, isem1, dsem0, dsem1)

            @pl.when(buf == 1)
            def _():
                halfstep(1, 0, gsem1, gsem0, isem1, isem0, dsem1, dsem0)

            return 0

        lax.fori_loop(0, nch, body, 0)

    @pl.when(cid == 0)
    def _():
        run(src0_hbm, dst0_hbm, F0)

    @pl.when(cid == 1)
    def _():
        run(src1_hbm, dst1_hbm, F1)

    plsc.subcore_barrier()

    pltpu.sync_copy(agg_sh.at[pl.ds(sid * RPT, RPT)],
                    p_hbm.at[cid, pl.ds(sid * RPT, RPT)])


def _mm_body(data_ref, wpre_ref, wpost_ref, h_ref):
    wc = jnp.dot(wpre_ref[...], wpost_ref[...], preferred_element_type=jnp.float32)
    h_ref[...] = jnp.dot(data_ref[...], wc, preferred_element_type=jnp.float32)


_mm_call = pl.pallas_call(
    _mm_body,
    out_shape=jax.ShapeDtypeStruct((N_NODES, D), jnp.float32),
)


def _scale_body(h_ref, degp_ref, hs_ref, isd_ref):
    deg = degp_ref[0] + degp_ref[1]
    isd = lax.rsqrt(jnp.maximum(deg, 1.0))
    isd_ref[...] = isd[:NPAD]
    hs_ref[0:N_NODES, :] = h_ref[...] * isd[:N_NODES]
    hs_ref[N_NODES:NPAD, :] = jnp.zeros((NPAD - N_NODES, D), jnp.float32)


_scale_call = pl.pallas_call(
    _scale_body,
    out_shape=(
        jax.ShapeDtypeStruct((NPAD, D), jnp.float32),
        jax.ShapeDtypeStruct((NPAD, 1), jnp.float32),
    ),
)


def _post_body(p_ref, isd_ref, out_ref):
    s = p_ref[0] + p_ref[1]
    out_ref[...] = s[:N_NODES] * isd_ref[0:N_NODES, :]


_post_call = pl.pallas_call(
    _post_body,
    out_shape=jax.ShapeDtypeStruct((N_NODES, D), jnp.float32),
)


@jax.jit
def kernel(data, edge_index, W_pre, W_post):
    src = edge_index[0]
    dst = edge_index[1]
    pad = EPAD - N_EDGES
    fill = jnp.full((pad,), N_NODES, jnp.int32)
    src_f = jnp.concatenate([src, fill])
    dst_f = jnp.concatenate([dst, fill])
    n0 = NS * F0 * CH
    src0 = src_f[:n0].reshape(NS, F0, CH)
    src1 = src_f[n0:].reshape(NS, F1, CH)
    dst0 = dst_f[:n0].reshape(NS, F0, CH)
    dst1 = dst_f[n0:].reshape(NS, F1, CH)

    degp = _deg_kernel(dst0, dst1, jnp.ones((CH,), jnp.float32),
                       jnp.zeros((NDEG,), jnp.float32))
    h = _mm_call(data, W_pre, W_post)
    degp_col = degp[:, :, None]
    hs, isd = _scale_call(h, degp_col)
    p = _scatter_kernel(hs, src0, src1, dst0, dst1)
    return _post_call(p, isd)


# speedup vs baseline: 1.1981x; 1.1981x over previous
import functools

import jax
import jax.numpy as jnp
from jax import lax
from jax.experimental import pallas as pl
from jax.experimental.pallas import tpu as pltpu
from jax.experimental.pallas import tpu_sc as plsc

N_NODES = 10000
N_EDGES = 320000
D = 128

NC = 2
NS = 16
NW = NC * NS
CH = 128
TOTCH = -(-N_EDGES // (NS * CH)) * NS
EPAD = TOTCH * CH

F0 = 79
F1 = TOTCH // NS - F0
FMAX = max(F0, F1)

NPAD = 10112
RPT = NPAD // NS
NDEG = 10240
DPT = NDEG // NS

_mesh = plsc.VectorSubcoreMesh(core_axis_name="c", subcore_axis_name="s")


@functools.partial(
    pl.kernel,
    out_type=jax.ShapeDtypeStruct((NC, NDEG), jnp.float32),
    mesh=_mesh,
    scratch_types=[
        pltpu.VMEM_SHARED((NDEG,), jnp.float32),
        pltpu.VMEM((FMAX, CH), jnp.int32),
        pltpu.VMEM((CH,), jnp.float32),
    ],
)
def _deg_kernel(dst0_hbm, dst1_hbm, ones_hbm, zeros_hbm, degp_hbm,
                deg_sh, dst_v, ones_v):
    cid = lax.axis_index("c")
    sid = lax.axis_index("s")

    pltpu.sync_copy(zeros_hbm.at[pl.ds(sid * DPT, DPT)],
                    deg_sh.at[pl.ds(sid * DPT, DPT)])
    pltpu.sync_copy(ones_hbm, ones_v)

    @pl.when(cid == 0)
    def _():
        pltpu.sync_copy(dst0_hbm.at[sid], dst_v.at[pl.ds(0, F0)])

    @pl.when(cid == 1)
    def _():
        pltpu.sync_copy(dst1_hbm.at[sid], dst_v.at[pl.ds(0, F1)])

    plsc.subcore_barrier()
    nch = jnp.where(cid == 0, F0, F1)

    def body(j, _):
        pltpu.sync_copy(ones_v, deg_sh.at[dst_v.at[j]], add=True)
        return 0

    lax.fori_loop(0, nch, body, 0)
    plsc.subcore_barrier()

    pltpu.sync_copy(deg_sh.at[pl.ds(sid * DPT, DPT)],
                    degp_hbm.at[cid, pl.ds(sid * DPT, DPT)])


@functools.partial(
    pl.kernel,
    out_type=jax.ShapeDtypeStruct((NC, NPAD, D), jnp.float32),
    mesh=_mesh,
    scratch_types=[
        pltpu.VMEM_SHARED((NPAD, D), jnp.float32),
        pltpu.VMEM((2, CH), jnp.int32),
        pltpu.VMEM((2, CH), jnp.int32),
        pltpu.VMEM((2, CH, D), jnp.float32),
        pltpu.SemaphoreType.DMA,
        pltpu.SemaphoreType.DMA,
        pltpu.SemaphoreType.DMA,
        pltpu.SemaphoreType.DMA,
        pltpu.SemaphoreType.DMA,
        pltpu.SemaphoreType.DMA,
    ],
)
def _scatter_kernel(hs_hbm, src0_hbm, src1_hbm, dst0_hbm, dst1_hbm, p_hbm,
                    agg_sh, sidx_v, didx_v, rows_v,
                    gsem0, gsem1, isem0, isem1, dsem0, dsem1):
    cid = lax.axis_index("c")
    sid = lax.axis_index("s")

    z16 = jnp.zeros((16,), jnp.float32)

    def zbody(r, _):
        for c in range(D // 16):
            rows_v[0, r, pl.ds(c * 16, 16)] = z16
        return 0

    lax.fori_loop(0, CH, zbody, 0)
    for k in range(4):
        pltpu.sync_copy(rows_v.at[0],
                        agg_sh.at[pl.ds(sid * RPT + k * CH, CH)])
    pltpu.sync_copy(rows_v.at[0, pl.ds(0, RPT - 4 * CH)],
                    agg_sh.at[pl.ds(sid * RPT + 4 * CH, RPT - 4 * CH)])

    plsc.subcore_barrier()

    def run(src_hbm, dst_hbm, nch):
        pltpu.sync_copy(src_hbm.at[sid, 0], sidx_v.at[0])
        pltpu.sync_copy(dst_hbm.at[sid, 0], didx_v.at[0])
        pltpu.async_copy(hs_hbm.at[sidx_v.at[0]], rows_v.at[0], gsem0)
        pltpu.async_copy(src_hbm.at[sid, 1], sidx_v.at[1], isem1)
        pltpu.async_copy(dst_hbm.at[sid, 1], didx_v.at[1], dsem1)

        def body(j, _):
            buf = lax.rem(j, 2)

            def halfstep(b, nb, gsem_b, gsem_nb, isem_b, isem_nb,
                         dsem_b, dsem_nb):
                @pl.when(j + 1 < nch)
                def _():
                    pltpu.make_async_copy(src_hbm.at[sid, j + 1], sidx_v.at[nb],
                                          isem_nb).wait()
                    pltpu.async_copy(hs_hbm.at[sidx_v.at[nb]], rows_v.at[nb],
                                     gsem_nb)

                pltpu.make_async_copy(hs_hbm.at[sidx_v.at[b]], rows_v.at[b],
                                      gsem_b).wait()

                @pl.when(j + 2 < nch)
                def _():
                    pltpu.async_copy(src_hbm.at[sid, j + 2], sidx_v.at[b],
                                     isem_b)

                @pl.when(j >= 1)
                def _():
                    pltpu.make_async_copy(dst_hbm.at[sid, j], didx_v.at[b],
                                          dsem_b).wait()

                pltpu.sync_copy(rows_v.at[b], agg_sh.at[didx_v.at[b]], add=True)

                @pl.when(j + 2 < nch)
                def _():
                    pltpu.async_copy(dst_hbm.at[sid, j + 2], didx_v.at[b],
                                     dsem_b)

            @pl.when(buf == 0)
            def _():
                halfstep(0, 1, gsem0, gsem1, isem0, isem1, dsem0, dsem1)

            @pl.when(buf == 1)
            def _():
                halfstep(1, 0, gsem1, gsem0, isem1, isem0, dsem1, dsem0)

            return 0

        lax.fori_loop(0, nch, body, 0)

    @pl.when(cid == 0)
    def _():
        run(src0_hbm, dst0_hbm, F0)

    @pl.when(cid == 1)
    def _():
        run(src1_hbm, dst1_hbm, F1)

    plsc.subcore_barrier()

    pltpu.sync_copy(agg_sh.at[pl.ds(sid * RPT, RPT)],
                    p_hbm.at[cid, pl.ds(sid * RPT, RPT)])


def _mm_body(data_ref, wpre_ref, wpost_ref, h_ref):
    wc = jnp.dot(wpre_ref[...], wpost_ref[...], preferred_element_type=jnp.float32)
    h_ref[...] = jnp.dot(data_ref[...], wc, preferred_element_type=jnp.float32)


_mm_call = pl.pallas_call(
    _mm_body,
    out_shape=jax.ShapeDtypeStruct((N_NODES, D), jnp.float32),
)


def _scale_body(h_ref, degp_ref, hs_ref, isd_ref):
    deg = degp_ref[0] + degp_ref[1]
    isd = lax.rsqrt(jnp.maximum(deg, 1.0))
    isd_ref[...] = isd[:NPAD]
    hs_ref[0:N_NODES, :] = h_ref[...] * isd[:N_NODES]
    hs_ref[N_NODES:NPAD, :] = jnp.zeros((NPAD - N_NODES, D), jnp.float32)


_scale_call = pl.pallas_call(
    _scale_body,
    out_shape=(
        jax.ShapeDtypeStruct((NPAD, D), jnp.float32),
        jax.ShapeDtypeStruct((NPAD, 1), jnp.float32),
    ),
)


def _post_body(p_ref, isd_ref, out_ref):
    s = p_ref[0] + p_ref[1]
    out_ref[...] = s[:N_NODES] * isd_ref[0:N_NODES, :]


_post_call = pl.pallas_call(
    _post_body,
    out_shape=jax.ShapeDtypeStruct((N_NODES, D), jnp.float32),
)


@jax.jit
def kernel(data, edge_index, W_pre, W_post):
    src = edge_index[0]
    dst = edge_index[1]
    pad = EPAD - N_EDGES
    fill = jnp.full((pad,), N_NODES, jnp.int32)
    src_f = jnp.concatenate([src, fill])
    dst_f = jnp.concatenate([dst, fill])
    n0 = NS * F0 * CH
    src0 = src_f[:n0].reshape(NS, F0, CH)
    src1 = src_f[n0:].reshape(NS, F1, CH)
    dst0 = dst_f[:n0].reshape(NS, F0, CH)
    dst1 = dst_f[n0:].reshape(NS, F1, CH)

    degp = _deg_kernel(dst0, dst1, jnp.ones((CH,), jnp.float32),
                       jnp.zeros((NDEG,), jnp.float32))
    h = _mm_call(data, W_pre, W_post)
    degp_col = degp[:, :, None]
    hs, isd = _scale_call(h, degp_col)
    p = _scatter_kernel(hs, src0, src1, dst0, dst1)
    return _post_call(p, isd)


# speedup vs baseline: 1.7398x; 1.4521x over previous
import functools

import jax
import jax.numpy as jnp
from jax import lax
from jax.experimental import pallas as pl
from jax.experimental.pallas import tpu as pltpu
from jax.experimental.pallas import tpu_sc as plsc

N_NODES = 10000
N_EDGES = 320000
D = 128

NC = 2
NS = 16
NW = NC * NS
CH = 128
NCHUNK = N_EDGES // CH
FBASE = NCHUNK // NW
FEXTRA = NCHUNK - FBASE * NW

NPAD = 10112
RPT = NPAD // NS
NDEG = 10240
DPT = NDEG // NS

_mesh = plsc.VectorSubcoreMesh(core_axis_name="c", subcore_axis_name="s")


def _tile_range(wid):
    start = wid * FBASE + jnp.minimum(wid, FEXTRA)
    n = jnp.where(wid < FEXTRA, FBASE + 1, FBASE)
    return start, n


@functools.partial(
    pl.kernel,
    out_type=jax.ShapeDtypeStruct((NC, NDEG), jnp.float32),
    mesh=_mesh,
    scratch_types=[
        pltpu.VMEM_SHARED((NDEG,), jnp.float32),
        pltpu.VMEM((FBASE + 1, CH), jnp.int32),
        pltpu.VMEM((CH,), jnp.float32),
        pltpu.SemaphoreType.DMA,
    ],
)
def _deg_kernel(dst_hbm, ones_hbm, zeros_hbm, degp_hbm, deg_sh, dst_v, ones_v,
                ssem):
    cid = lax.axis_index("c")
    sid = lax.axis_index("s")
    wid = cid * NS + sid
    start, nch = _tile_range(wid)

    pltpu.sync_copy(zeros_hbm.at[pl.ds(sid * DPT, DPT)],
                    deg_sh.at[pl.ds(sid * DPT, DPT)])
    pltpu.sync_copy(ones_hbm, ones_v)

    def sbody(j, _):
        pltpu.async_copy(dst_hbm.at[start + j], dst_v.at[j], ssem)
        return 0

    lax.fori_loop(0, nch, sbody, 0)

    def wbody(j, _):
        pltpu.make_async_copy(dst_hbm.at[start + j], dst_v.at[j], ssem).wait()
        return 0

    lax.fori_loop(0, nch, wbody, 0)
    plsc.subcore_barrier()

    def body(j, _):
        pltpu.sync_copy(ones_v, deg_sh.at[dst_v.at[j]], add=True)
        return 0

    lax.fori_loop(0, nch, body, 0)
    plsc.subcore_barrier()

    pltpu.sync_copy(deg_sh.at[pl.ds(sid * DPT, DPT)],
                    degp_hbm.at[cid, pl.ds(sid * DPT, DPT)])


@functools.partial(
    pl.kernel,
    out_type=jax.ShapeDtypeStruct((NC, NPAD, D), jnp.float32),
    mesh=_mesh,
    scratch_types=[
        pltpu.VMEM_SHARED((NPAD, D), jnp.float32),
        pltpu.VMEM((2, CH), jnp.int32),
        pltpu.VMEM((2, CH), jnp.int32),
        pltpu.VMEM((2, CH, D), jnp.float32),
        pltpu.SemaphoreType.DMA,
        pltpu.SemaphoreType.DMA,
        pltpu.SemaphoreType.DMA,
        pltpu.SemaphoreType.DMA,
        pltpu.SemaphoreType.DMA,
        pltpu.SemaphoreType.DMA,
    ],
)
def _scatter_kernel(hs_hbm, src_hbm, dst_hbm, p_hbm,
                    agg_sh, sidx_v, didx_v, rows_v,
                    gsem0, gsem1, isem0, isem1, dsem0, dsem1):
    cid = lax.axis_index("c")
    sid = lax.axis_index("s")
    wid = cid * NS + sid
    start, nch = _tile_range(wid)

    z16 = jnp.zeros((16,), jnp.float32)

    def zbody(r, _):
        for c in range(D // 16):
            rows_v[0, r, pl.ds(c * 16, 16)] = z16
        return 0

    lax.fori_loop(0, CH, zbody, 0)
    for k in range(4):
        pltpu.sync_copy(rows_v.at[0],
                        agg_sh.at[pl.ds(sid * RPT + k * CH, CH)])
    pltpu.sync_copy(rows_v.at[0, pl.ds(0, RPT - 4 * CH)],
                    agg_sh.at[pl.ds(sid * RPT + 4 * CH, RPT - 4 * CH)])

    plsc.subcore_barrier()

    pltpu.sync_copy(src_hbm.at[start + 0], sidx_v.at[0])
    pltpu.sync_copy(dst_hbm.at[start + 0], didx_v.at[0])
    pltpu.async_copy(hs_hbm.at[sidx_v.at[0]], rows_v.at[0], gsem0)
    pltpu.async_copy(src_hbm.at[start + 1], sidx_v.at[1], isem1)
    pltpu.async_copy(dst_hbm.at[start + 1], didx_v.at[1], dsem1)

    def body(j, _):
        buf = lax.rem(j, 2)

        def halfstep(b, nb, gsem_b, gsem_nb, isem_b, isem_nb, dsem_b, dsem_nb):
            @pl.when(j + 1 < nch)
            def _():
                pltpu.make_async_copy(src_hbm.at[start + j + 1], sidx_v.at[nb],
                                      isem_nb).wait()
                pltpu.async_copy(hs_hbm.at[sidx_v.at[nb]], rows_v.at[nb],
                                 gsem_nb)

            pltpu.make_async_copy(hs_hbm.at[sidx_v.at[b]], rows_v.at[b],
                                  gsem_b).wait()

            @pl.when(j + 2 < nch)
            def _():
                pltpu.async_copy(src_hbm.at[start + j + 2], sidx_v.at[b],
                                 isem_b)

            @pl.when(j >= 1)
            def _():
                pltpu.make_async_copy(dst_hbm.at[start + j], didx_v.at[b],
                                      dsem_b).wait()

            pltpu.sync_copy(rows_v.at[b], agg_sh.at[didx_v.at[b]], add=True)

            @pl.when(j + 2 < nch)
            def _():
                pltpu.async_copy(dst_hbm.at[start + j + 2], didx_v.at[b],
                                 dsem_b)

        @pl.when(buf == 0)
        def _():
            halfstep(0, 1, gsem0, gsem1, isem0, isem1, dsem0, dsem1)

        @pl.when(buf == 1)
        def _():
            halfstep(1, 0, gsem1, gsem0, isem1, isem0, dsem1, dsem0)

        return 0

    lax.fori_loop(0, nch, body, 0)
    plsc.subcore_barrier()

    pltpu.sync_copy(agg_sh.at[pl.ds(sid * RPT, RPT)],
                    p_hbm.at[cid, pl.ds(sid * RPT, RPT)])


def _mm_body(data_ref, wpre_ref, wpost_ref, h_ref):
    wc = jnp.dot(wpre_ref[...], wpost_ref[...], preferred_element_type=jnp.float32)
    h_ref[...] = jnp.dot(data_ref[...], wc, preferred_element_type=jnp.float32)


_mm_call = pl.pallas_call(
    _mm_body,
    out_shape=jax.ShapeDtypeStruct((N_NODES, D), jnp.float32),
)


def _scale_body(h_ref, degp_ref, hs_ref, isd_ref):
    deg = degp_ref[0] + degp_ref[1]
    isd = lax.rsqrt(jnp.maximum(deg, 1.0))
    isd_ref[...] = isd[:NPAD]
    hs_ref[0:N_NODES, :] = h_ref[...] * isd[:N_NODES]
    hs_ref[N_NODES:NPAD, :] = jnp.zeros((NPAD - N_NODES, D), jnp.float32)


_scale_call = pl.pallas_call(
    _scale_body,
    out_shape=(
        jax.ShapeDtypeStruct((NPAD, D), jnp.float32),
        jax.ShapeDtypeStruct((NPAD, 1), jnp.float32),
    ),
)


def _post_body(p_ref, isd_ref, out_ref):
    s = p_ref[0] + p_ref[1]
    out_ref[...] = s[:N_NODES] * isd_ref[0:N_NODES, :]


_post_call = pl.pallas_call(
    _post_body,
    out_shape=jax.ShapeDtypeStruct((N_NODES, D), jnp.float32),
)


@jax.jit
def kernel(data, edge_index, W_pre, W_post):
    src = edge_index[0].reshape(NCHUNK, CH)
    dst = edge_index[1].reshape(NCHUNK, CH)

    degp = _deg_kernel(dst, jnp.ones((CH,), jnp.float32),
                       jnp.zeros((NDEG,), jnp.float32))
    h = _mm_call(data, W_pre, W_post)
    degp_col = degp[:, :, None]
    hs, isd = _scale_call(h, degp_col)
    p = _scatter_kernel(hs, src, dst)
    return _post_call(p, isd)


# speedup vs baseline: 2.0178x; 1.1598x over previous
import functools

import jax
import jax.numpy as jnp
from jax import lax
from jax.experimental import pallas as pl
from jax.experimental.pallas import tpu as pltpu
from jax.experimental.pallas import tpu_sc as plsc

N_NODES = 10000
N_EDGES = 320000
D = 128

NC = 2
NS = 16
NW = NC * NS
CH = 128
NCHUNK = N_EDGES // CH
FBASE = NCHUNK // NW
FEXTRA = NCHUNK - FBASE * NW

NPAD = 10112
RPT = NPAD // NS
NDEG = 10240
DPT = NDEG // NS

_mesh = plsc.VectorSubcoreMesh(core_axis_name="c", subcore_axis_name="s")


def _tile_range(wid):
    start = wid * FBASE + jnp.minimum(wid, FEXTRA)
    n = jnp.where(wid < FEXTRA, FBASE + 1, FBASE)
    return start, n


@functools.partial(
    pl.kernel,
    out_type=jax.ShapeDtypeStruct((NC, NDEG), jnp.float32),
    mesh=_mesh,
    scratch_types=[
        pltpu.VMEM_SHARED((NDEG,), jnp.float32),
        pltpu.VMEM((FBASE + 1, CH), jnp.int32),
        pltpu.VMEM((CH,), jnp.float32),
        pltpu.SemaphoreType.DMA,
    ],
)
def _deg_kernel(ei_hbm, ones_hbm, zeros_hbm, degp_hbm, deg_sh, dst_v, ones_v,
                ssem):
    cid = lax.axis_index("c")
    sid = lax.axis_index("s")
    wid = cid * NS + sid
    start, nch = _tile_range(wid)

    pltpu.sync_copy(zeros_hbm.at[pl.ds(sid * DPT, DPT)],
                    deg_sh.at[pl.ds(sid * DPT, DPT)])
    pltpu.sync_copy(ones_hbm, ones_v)

    def sbody(j, _):
        pltpu.async_copy(ei_hbm.at[1, start + j], dst_v.at[j], ssem)
        return 0

    lax.fori_loop(0, nch, sbody, 0)

    def wbody(j, _):
        pltpu.make_async_copy(ei_hbm.at[1, start + j], dst_v.at[j], ssem).wait()
        return 0

    lax.fori_loop(0, nch, wbody, 0)
    plsc.subcore_barrier()

    def body(j, _):
        pltpu.sync_copy(ones_v, deg_sh.at[dst_v.at[j]], add=True)
        return 0

    lax.fori_loop(0, nch, body, 0)
    plsc.subcore_barrier()

    pltpu.sync_copy(deg_sh.at[pl.ds(sid * DPT, DPT)],
                    degp_hbm.at[cid, pl.ds(sid * DPT, DPT)])


@functools.partial(
    pl.kernel,
    out_type=jax.ShapeDtypeStruct((NC, NPAD, D), jnp.float32),
    mesh=_mesh,
    scratch_types=[
        pltpu.VMEM_SHARED((NPAD, D), jnp.float32),
        pltpu.VMEM((2, CH), jnp.int32),
        pltpu.VMEM((2, CH), jnp.int32),
        pltpu.VMEM((2, CH, D), jnp.float32),
        pltpu.SemaphoreType.DMA,
        pltpu.SemaphoreType.DMA,
        pltpu.SemaphoreType.DMA,
        pltpu.SemaphoreType.DMA,
        pltpu.SemaphoreType.DMA,
        pltpu.SemaphoreType.DMA,
    ],
)
def _scatter_kernel(hs_hbm, ei_hbm, p_hbm,
                    agg_sh, sidx_v, didx_v, rows_v,
                    gsem0, gsem1, isem0, isem1, dsem0, dsem1):
    cid = lax.axis_index("c")
    sid = lax.axis_index("s")
    wid = cid * NS + sid
    start, nch = _tile_range(wid)

    z16 = jnp.zeros((16,), jnp.float32)

    def zbody(r, _):
        for c in range(D // 16):
            rows_v[0, r, pl.ds(c * 16, 16)] = z16
        return 0

    lax.fori_loop(0, CH, zbody, 0)
    for k in range(4):
        pltpu.sync_copy(rows_v.at[0],
                        agg_sh.at[pl.ds(sid * RPT + k * CH, CH)])
    pltpu.sync_copy(rows_v.at[0, pl.ds(0, RPT - 4 * CH)],
                    agg_sh.at[pl.ds(sid * RPT + 4 * CH, RPT - 4 * CH)])

    plsc.subcore_barrier()

    pltpu.sync_copy(ei_hbm.at[0, start + 0], sidx_v.at[0])
    pltpu.sync_copy(ei_hbm.at[1, start + 0], didx_v.at[0])
    pltpu.async_copy(hs_hbm.at[sidx_v.at[0]], rows_v.at[0], gsem0)
    pltpu.async_copy(ei_hbm.at[0, start + 1], sidx_v.at[1], isem1)
    pltpu.async_copy(ei_hbm.at[1, start + 1], didx_v.at[1], dsem1)

    def body(j, _):
        buf = lax.rem(j, 2)

        def halfstep(b, nb, gsem_b, gsem_nb, isem_b, isem_nb, dsem_b, dsem_nb):
            @pl.when(j + 1 < nch)
            def _():
                pltpu.make_async_copy(ei_hbm.at[0, start + j + 1], sidx_v.at[nb],
                                      isem_nb).wait()
                pltpu.async_copy(hs_hbm.at[sidx_v.at[nb]], rows_v.at[nb],
                                 gsem_nb)

            pltpu.make_async_copy(hs_hbm.at[sidx_v.at[b]], rows_v.at[b],
                                  gsem_b).wait()

            @pl.when(j + 2 < nch)
            def _():
                pltpu.async_copy(ei_hbm.at[0, start + j + 2], sidx_v.at[b],
                                 isem_b)

            @pl.when(j >= 1)
            def _():
                pltpu.make_async_copy(ei_hbm.at[1, start + j], didx_v.at[b],
                                      dsem_b).wait()

            pltpu.sync_copy(rows_v.at[b], agg_sh.at[didx_v.at[b]], add=True)

            @pl.when(j + 2 < nch)
            def _():
                pltpu.async_copy(ei_hbm.at[1, start + j + 2], didx_v.at[b],
                                 dsem_b)

        @pl.when(buf == 0)
        def _():
            halfstep(0, 1, gsem0, gsem1, isem0, isem1, dsem0, dsem1)

        @pl.when(buf == 1)
        def _():
            halfstep(1, 0, gsem1, gsem0, isem1, isem0, dsem1, dsem0)

        return 0

    lax.fori_loop(0, nch, body, 0)
    plsc.subcore_barrier()

    pltpu.sync_copy(agg_sh.at[pl.ds(sid * RPT, RPT)],
                    p_hbm.at[cid, pl.ds(sid * RPT, RPT)])


def _mm_body(data_ref, wpre_ref, wpost_ref, h_ref):
    wc = jnp.dot(wpre_ref[...], wpost_ref[...], preferred_element_type=jnp.float32)
    h_ref[...] = jnp.dot(data_ref[...], wc, preferred_element_type=jnp.float32)


_mm_call = pl.pallas_call(
    _mm_body,
    out_shape=jax.ShapeDtypeStruct((N_NODES, D), jnp.float32),
)


def _isd_col(degp, n):
    deg = degp[0] + degp[1]
    isd = lax.rsqrt(jnp.maximum(deg, 1.0))
    return isd[:n, None]


def _scale_body(h_ref, degp_ref, hs_ref):
    hs_ref[0:N_NODES, :] = h_ref[...] * _isd_col(degp_ref[...], N_NODES)
    hs_ref[N_NODES:NPAD, :] = jnp.zeros((NPAD - N_NODES, D), jnp.float32)


_scale_call = pl.pallas_call(
    _scale_body,
    out_shape=jax.ShapeDtypeStruct((NPAD, D), jnp.float32),
)


def _post_body(p_ref, degp_ref, out_ref):
    s = p_ref[0] + p_ref[1]
    out_ref[...] = s[:N_NODES] * _isd_col(degp_ref[...], N_NODES)


_post_call = pl.pallas_call(
    _post_body,
    out_shape=jax.ShapeDtypeStruct((N_NODES, D), jnp.float32),
)


@jax.jit
def kernel(data, edge_index, W_pre, W_post):
    ei = edge_index.reshape(2, NCHUNK, CH)

    degp = _deg_kernel(ei, jnp.ones((CH,), jnp.float32),
                       jnp.zeros((NDEG,), jnp.float32))
    h = _mm_call(data, W_pre, W_post)
    hs = _scale_call(h, degp)
    p = _scatter_kernel(hs, ei)
    return _post_call(p, degp)


# speedup vs baseline: 2.0807x; 1.0312x over previous
import functools

import jax
import jax.numpy as jnp
from jax import lax
from jax.experimental import pallas as pl
from jax.experimental.pallas import tpu as pltpu
from jax.experimental.pallas import tpu_sc as plsc

N_NODES = 10000
N_EDGES = 320000
D = 128

NC = 2
NS = 16
NW = NC * NS
CH = 128
NCHUNK = N_EDGES // CH
FBASE = NCHUNK // NW
FEXTRA = NCHUNK - FBASE * NW

NPAD = 10112
RPT = NPAD // NS
NDEG = 10240
DPT = NDEG // NS

_mesh = plsc.VectorSubcoreMesh(core_axis_name="c", subcore_axis_name="s")


def _tile_range(wid):
    start = wid * FBASE + jnp.minimum(wid, FEXTRA)
    n = jnp.where(wid < FEXTRA, FBASE + 1, FBASE)
    return start, n


@functools.partial(
    pl.kernel,
    out_type=jax.ShapeDtypeStruct((NC, NDEG), jnp.float32),
    mesh=_mesh,
    scratch_types=[
        pltpu.VMEM_SHARED((NDEG,), jnp.float32),
        pltpu.VMEM((FBASE + 1, CH), jnp.int32),
        pltpu.VMEM((CH,), jnp.float32),
        pltpu.SemaphoreType.DMA,
    ],
)
def _deg_kernel(ei_hbm, ones_hbm, zeros_hbm, degp_hbm, deg_sh, dst_v, ones_v,
                ssem):
    cid = lax.axis_index("c")
    sid = lax.axis_index("s")
    wid = cid * NS + sid
    start, nch = _tile_range(wid)

    pltpu.sync_copy(zeros_hbm.at[pl.ds(sid * DPT, DPT)],
                    deg_sh.at[pl.ds(sid * DPT, DPT)])
    pltpu.sync_copy(ones_hbm, ones_v)

    def sbody(j, _):
        pltpu.async_copy(ei_hbm.at[1, pl.ds((start + j) * CH, CH)],
                         dst_v.at[j], ssem)
        return 0

    lax.fori_loop(0, nch, sbody, 0)

    def wbody(j, _):
        pltpu.make_async_copy(ei_hbm.at[1, pl.ds((start + j) * CH, CH)],
                              dst_v.at[j], ssem).wait()
        return 0

    lax.fori_loop(0, nch, wbody, 0)
    plsc.subcore_barrier()

    def body(j, _):
        pltpu.sync_copy(ones_v, deg_sh.at[dst_v.at[j]], add=True)
        return 0

    lax.fori_loop(0, nch, body, 0)
    plsc.subcore_barrier()

    pltpu.sync_copy(deg_sh.at[pl.ds(sid * DPT, DPT)],
                    degp_hbm.at[cid, pl.ds(sid * DPT, DPT)])


@functools.partial(
    pl.kernel,
    out_type=jax.ShapeDtypeStruct((NC, NPAD, D), jnp.float32),
    mesh=_mesh,
    scratch_types=[
        pltpu.VMEM_SHARED((NPAD, D), jnp.float32),
        pltpu.VMEM((2, CH), jnp.int32),
        pltpu.VMEM((2, CH), jnp.int32),
        pltpu.VMEM((2, CH, D), jnp.float32),
        pltpu.SemaphoreType.DMA,
        pltpu.SemaphoreType.DMA,
        pltpu.SemaphoreType.DMA,
        pltpu.SemaphoreType.DMA,
        pltpu.SemaphoreType.DMA,
        pltpu.SemaphoreType.DMA,
    ],
)
def _scatter_kernel(hs_hbm, ei_hbm, p_hbm,
                    agg_sh, sidx_v, didx_v, rows_v,
                    gsem0, gsem1, isem0, isem1, dsem0, dsem1):
    cid = lax.axis_index("c")
    sid = lax.axis_index("s")
    wid = cid * NS + sid
    start, nch = _tile_range(wid)

    pltpu.sync_copy(ei_hbm.at[0, pl.ds((start + 0) * CH, CH)], sidx_v.at[0])
    pltpu.sync_copy(ei_hbm.at[1, pl.ds((start + 0) * CH, CH)], didx_v.at[0])
    pltpu.async_copy(hs_hbm.at[sidx_v.at[0]], rows_v.at[0], gsem0)
    pltpu.async_copy(ei_hbm.at[0, pl.ds((start + 1) * CH, CH)], sidx_v.at[1], isem1)
    pltpu.async_copy(ei_hbm.at[1, pl.ds((start + 1) * CH, CH)], didx_v.at[1], dsem1)

    z16 = jnp.zeros((16,), jnp.float32)

    def zbody(r, _):
        for c in range(D // 16):
            rows_v[1, r, pl.ds(c * 16, 16)] = z16
        return 0

    lax.fori_loop(0, CH, zbody, 0)
    for k in range(4):
        pltpu.sync_copy(rows_v.at[1],
                        agg_sh.at[pl.ds(sid * RPT + k * CH, CH)])
    pltpu.sync_copy(rows_v.at[1, pl.ds(0, RPT - 4 * CH)],
                    agg_sh.at[pl.ds(sid * RPT + 4 * CH, RPT - 4 * CH)])

    plsc.subcore_barrier()

    def body(j, _):
        buf = lax.rem(j, 2)

        def halfstep(b, nb, gsem_b, gsem_nb, isem_b, isem_nb, dsem_b, dsem_nb):
            @pl.when(j + 1 < nch)
            def _():
                pltpu.make_async_copy(ei_hbm.at[0, pl.ds((start + j + 1) * CH, CH)], sidx_v.at[nb],
                                      isem_nb).wait()
                pltpu.async_copy(hs_hbm.at[sidx_v.at[nb]], rows_v.at[nb],
                                 gsem_nb)

            pltpu.make_async_copy(hs_hbm.at[sidx_v.at[b]], rows_v.at[b],
                                  gsem_b).wait()

            @pl.when(j + 2 < nch)
            def _():
                pltpu.async_copy(ei_hbm.at[0, pl.ds((start + j + 2) * CH, CH)], sidx_v.at[b],
                                 isem_b)

            @pl.when(j >= 1)
            def _():
                pltpu.make_async_copy(ei_hbm.at[1, pl.ds((start + j) * CH, CH)], didx_v.at[b],
                                      dsem_b).wait()

            pltpu.sync_copy(rows_v.at[b], agg_sh.at[didx_v.at[b]], add=True)

            @pl.when(j + 2 < nch)
            def _():
                pltpu.async_copy(ei_hbm.at[1, pl.ds((start + j + 2) * CH, CH)], didx_v.at[b],
                                 dsem_b)

        @pl.when(buf == 0)
        def _():
            halfstep(0, 1, gsem0, gsem1, isem0, isem1, dsem0, dsem1)

        @pl.when(buf == 1)
        def _():
            halfstep(1, 0, gsem1, gsem0, isem1, isem0, dsem1, dsem0)

        return 0

    lax.fori_loop(0, nch, body, 0)
    plsc.subcore_barrier()

    pltpu.sync_copy(agg_sh.at[pl.ds(sid * RPT, RPT)],
                    p_hbm.at[cid, pl.ds(sid * RPT, RPT)])


def _mm_body(data_ref, wpre_ref, wpost_ref, h_ref):
    wc = jnp.dot(wpre_ref[...], wpost_ref[...], preferred_element_type=jnp.float32)
    h_ref[...] = jnp.dot(data_ref[...], wc, preferred_element_type=jnp.float32)


_mm_call = pl.pallas_call(
    _mm_body,
    out_shape=jax.ShapeDtypeStruct((N_NODES, D), jnp.float32),
)


def _isd_col(degp, n):
    deg = degp[0] + degp[1]
    isd = lax.rsqrt(jnp.maximum(deg, 1.0))
    return isd[:n, None]


def _scale_body(h_ref, degp_ref, hs_ref):
    hs_ref[0:N_NODES, :] = h_ref[...] * _isd_col(degp_ref[...], N_NODES)
    hs_ref[N_NODES:NPAD, :] = jnp.zeros((NPAD - N_NODES, D), jnp.float32)


_scale_call = pl.pallas_call(
    _scale_body,
    out_shape=jax.ShapeDtypeStruct((NPAD, D), jnp.float32),
)


def _post_body(p_ref, degp_ref, out_ref):
    s = p_ref[0] + p_ref[1]
    out_ref[...] = s[:N_NODES] * _isd_col(degp_ref[...], N_NODES)


_post_call = pl.pallas_call(
    _post_body,
    out_shape=jax.ShapeDtypeStruct((N_NODES, D), jnp.float32),
)


@jax.jit
def kernel(data, edge_index, W_pre, W_post):
    degp = _deg_kernel(edge_index, jnp.ones((CH,), jnp.float32),
                       jnp.zeros((NDEG,), jnp.float32))
    h = _mm_call(data, W_pre, W_post)
    hs = _scale_call(h, degp)
    p = _scatter_kernel(hs, edge_index)
    return _post_call(p, degp)


# speedup vs baseline: 2.1271x; 1.0223x over previous
import functools

import jax
import jax.numpy as jnp
from jax import lax
from jax.experimental import pallas as pl
from jax.experimental.pallas import tpu as pltpu
from jax.experimental.pallas import tpu_sc as plsc

N_NODES = 10000
N_EDGES = 320000
D = 128

NC = 2
NS = 16
NW = NC * NS
CH = 128
NCHUNK = N_EDGES // CH
FBASE = NCHUNK // NW
FEXTRA = NCHUNK - FBASE * NW

NPAD = 10112
RPT = NPAD // NS
NDEG = 10240
DPT = NDEG // NS

_mesh = plsc.VectorSubcoreMesh(core_axis_name="c", subcore_axis_name="s")


def _tile_range(wid):
    start = wid * FBASE + jnp.minimum(wid, FEXTRA)
    n = jnp.where(wid < FEXTRA, FBASE + 1, FBASE)
    return start, n


@functools.partial(
    pl.kernel,
    out_type=jax.ShapeDtypeStruct((NC, NDEG), jnp.float32),
    mesh=_mesh,
    scratch_types=[
        pltpu.VMEM_SHARED((NDEG,), jnp.float32),
        pltpu.VMEM((FBASE + 1, CH), jnp.int32),
        pltpu.VMEM((CH,), jnp.float32),
        pltpu.VMEM((DPT,), jnp.float32),
        pltpu.SemaphoreType.DMA,
    ],
)
def _deg_kernel(ei_hbm, degp_hbm, deg_sh, dst_v, ones_v, zb_v, ssem):
    cid = lax.axis_index("c")
    sid = lax.axis_index("s")
    wid = cid * NS + sid
    start, nch = _tile_range(wid)

    def sbody(j, _):
        pltpu.async_copy(ei_hbm.at[1, pl.ds((start + j) * CH, CH)],
                         dst_v.at[j], ssem)
        return 0

    lax.fori_loop(0, nch, sbody, 0)

    one16 = jnp.ones((16,), jnp.float32)
    z16 = jnp.zeros((16,), jnp.float32)
    for c in range(CH // 16):
        ones_v[pl.ds(c * 16, 16)] = one16

    def zvbody(c, _):
        zb_v[pl.ds(c * 16, 16)] = z16
        return 0

    lax.fori_loop(0, DPT // 16, zvbody, 0)
    pltpu.sync_copy(zb_v, deg_sh.at[pl.ds(sid * DPT, DPT)])
    plsc.subcore_barrier()

    def body(j, _):
        pltpu.make_async_copy(ei_hbm.at[1, pl.ds((start + j) * CH, CH)],
                              dst_v.at[j], ssem).wait()
        pltpu.sync_copy(ones_v, deg_sh.at[dst_v.at[j]], add=True)
        return 0

    lax.fori_loop(0, nch, body, 0)
    plsc.subcore_barrier()

    pltpu.sync_copy(deg_sh.at[pl.ds(sid * DPT, DPT)],
                    degp_hbm.at[cid, pl.ds(sid * DPT, DPT)])


@functools.partial(
    pl.kernel,
    out_type=jax.ShapeDtypeStruct((NC, NPAD, D), jnp.float32),
    mesh=_mesh,
    scratch_types=[
        pltpu.VMEM_SHARED((NPAD, D), jnp.float32),
        pltpu.VMEM((2, CH), jnp.int32),
        pltpu.VMEM((2, CH), jnp.int32),
        pltpu.VMEM((2, CH, D), jnp.float32),
        pltpu.SemaphoreType.DMA,
        pltpu.SemaphoreType.DMA,
        pltpu.SemaphoreType.DMA,
        pltpu.SemaphoreType.DMA,
        pltpu.SemaphoreType.DMA,
        pltpu.SemaphoreType.DMA,
    ],
)
def _scatter_kernel(hs_hbm, ei_hbm, p_hbm,
                    agg_sh, sidx_v, didx_v, rows_v,
                    gsem0, gsem1, isem0, isem1, dsem0, dsem1):
    cid = lax.axis_index("c")
    sid = lax.axis_index("s")
    wid = cid * NS + sid
    start, nch = _tile_range(wid)

    pltpu.sync_copy(ei_hbm.at[0, pl.ds((start + 0) * CH, CH)], sidx_v.at[0])
    pltpu.sync_copy(ei_hbm.at[1, pl.ds((start + 0) * CH, CH)], didx_v.at[0])
    pltpu.async_copy(hs_hbm.at[sidx_v.at[0]], rows_v.at[0], gsem0)
    pltpu.async_copy(ei_hbm.at[0, pl.ds((start + 1) * CH, CH)], sidx_v.at[1], isem1)
    pltpu.async_copy(ei_hbm.at[1, pl.ds((start + 1) * CH, CH)], didx_v.at[1], dsem1)

    z16 = jnp.zeros((16,), jnp.float32)

    def zbody(r, _):
        for c in range(D // 16):
            rows_v[1, r, pl.ds(c * 16, 16)] = z16
        return 0

    lax.fori_loop(0, CH, zbody, 0)
    for k in range(4):
        pltpu.sync_copy(rows_v.at[1],
                        agg_sh.at[pl.ds(sid * RPT + k * CH, CH)])
    pltpu.sync_copy(rows_v.at[1, pl.ds(0, RPT - 4 * CH)],
                    agg_sh.at[pl.ds(sid * RPT + 4 * CH, RPT - 4 * CH)])

    plsc.subcore_barrier()

    def body(j, _):
        buf = lax.rem(j, 2)

        def halfstep(b, nb, gsem_b, gsem_nb, isem_b, isem_nb, dsem_b, dsem_nb):
            @pl.when(j + 1 < nch)
            def _():
                pltpu.make_async_copy(ei_hbm.at[0, pl.ds((start + j + 1) * CH, CH)], sidx_v.at[nb],
                                      isem_nb).wait()
                pltpu.async_copy(hs_hbm.at[sidx_v.at[nb]], rows_v.at[nb],
                                 gsem_nb)

            pltpu.make_async_copy(hs_hbm.at[sidx_v.at[b]], rows_v.at[b],
                                  gsem_b).wait()

            @pl.when(j + 2 < nch)
            def _():
                pltpu.async_copy(ei_hbm.at[0, pl.ds((start + j + 2) * CH, CH)], sidx_v.at[b],
                                 isem_b)

            @pl.when(j >= 1)
            def _():
                pltpu.make_async_copy(ei_hbm.at[1, pl.ds((start + j) * CH, CH)], didx_v.at[b],
                                      dsem_b).wait()

            pltpu.sync_copy(rows_v.at[b], agg_sh.at[didx_v.at[b]], add=True)

            @pl.when(j + 2 < nch)
            def _():
                pltpu.async_copy(ei_hbm.at[1, pl.ds((start + j + 2) * CH, CH)], didx_v.at[b],
                                 dsem_b)

        @pl.when(buf == 0)
        def _():
            halfstep(0, 1, gsem0, gsem1, isem0, isem1, dsem0, dsem1)

        @pl.when(buf == 1)
        def _():
            halfstep(1, 0, gsem1, gsem0, isem1, isem0, dsem1, dsem0)

        return 0

    lax.fori_loop(0, nch, body, 0)
    plsc.subcore_barrier()

    pltpu.sync_copy(agg_sh.at[pl.ds(sid * RPT, RPT)],
                    p_hbm.at[cid, pl.ds(sid * RPT, RPT)])


def _mm_body(data_ref, wpre_ref, wpost_ref, h_ref):
    wc = jnp.dot(wpre_ref[...], wpost_ref[...], preferred_element_type=jnp.float32)
    h_ref[...] = jnp.dot(data_ref[...], wc, preferred_element_type=jnp.float32)


_mm_call = pl.pallas_call(
    _mm_body,
    out_shape=jax.ShapeDtypeStruct((N_NODES, D), jnp.float32),
)


def _isd_col(degp, n):
    deg = degp[0] + degp[1]
    isd = lax.rsqrt(jnp.maximum(deg, 1.0))
    return isd[:n, None]


def _scale_body(h_ref, degp_ref, hs_ref):
    hs_ref[0:N_NODES, :] = h_ref[...] * _isd_col(degp_ref[...], N_NODES)
    hs_ref[N_NODES:NPAD, :] = jnp.zeros((NPAD - N_NODES, D), jnp.float32)


_scale_call = pl.pallas_call(
    _scale_body,
    out_shape=jax.ShapeDtypeStruct((NPAD, D), jnp.float32),
)


def _post_body(p_ref, degp_ref, out_ref):
    s = p_ref[0] + p_ref[1]
    out_ref[...] = s[:N_NODES] * _isd_col(degp_ref[...], N_NODES)


_post_call = pl.pallas_call(
    _post_body,
    out_shape=jax.ShapeDtypeStruct((N_NODES, D), jnp.float32),
)


@jax.jit
def kernel(data, edge_index, W_pre, W_post):
    degp = _deg_kernel(edge_index)
    h = _mm_call(data, W_pre, W_post)
    hs = _scale_call(h, degp)
    p = _scatter_kernel(hs, edge_index)
    return _post_call(p, degp)


# speedup vs baseline: 2.1382x; 1.0052x over previous
import functools

import jax
import jax.numpy as jnp
from jax import lax
from jax.experimental import pallas as pl
from jax.experimental.pallas import tpu as pltpu
from jax.experimental.pallas import tpu_sc as plsc

N_NODES = 10000
N_EDGES = 320000
D = 128

NC = 2
NS = 16
NW = NC * NS
CH = 128
NCHUNK = N_EDGES // CH
FBASE = NCHUNK // NW
FEXTRA = NCHUNK - FBASE * NW

NPAD = 10112
RPT = NPAD // NS
NDEG = 10240
DPT = NDEG // NS

_mesh = plsc.VectorSubcoreMesh(core_axis_name="c", subcore_axis_name="s")


def _tile_range(wid):
    start = wid * FBASE + jnp.minimum(wid, FEXTRA)
    n = jnp.where(wid < FEXTRA, FBASE + 1, FBASE)
    return start, n


@functools.partial(
    pl.kernel,
    out_type=jax.ShapeDtypeStruct((NC, NDEG), jnp.float32),
    mesh=_mesh,
    scratch_types=[
        pltpu.VMEM_SHARED((NDEG,), jnp.float32),
        pltpu.VMEM((FBASE + 1, CH), jnp.int32),
        pltpu.VMEM((CH,), jnp.float32),
        pltpu.VMEM((DPT,), jnp.float32),
        pltpu.SemaphoreType.DMA,
    ],
)
def _deg_kernel(ei_hbm, degp_hbm, deg_sh, dst_v, ones_v, zb_v, ssem):
    cid = lax.axis_index("c")
    sid = lax.axis_index("s")
    wid = cid * NS + sid
    start, nch = _tile_range(wid)

    def sbody(j, _):
        pltpu.async_copy(ei_hbm.at[1, pl.ds((start + j) * CH, CH)],
                         dst_v.at[j], ssem)
        return 0

    lax.fori_loop(0, nch, sbody, 0)

    one16 = jnp.ones((16,), jnp.float32)
    z16 = jnp.zeros((16,), jnp.float32)
    for c in range(CH // 16):
        ones_v[pl.ds(c * 16, 16)] = one16

    def zvbody(c, _):
        zb_v[pl.ds(c * 16, 16)] = z16
        return 0

    lax.fori_loop(0, DPT // 16, zvbody, 0)
    pltpu.sync_copy(zb_v, deg_sh.at[pl.ds(sid * DPT, DPT)])
    plsc.subcore_barrier()

    def body(j, _):
        pltpu.make_async_copy(ei_hbm.at[1, pl.ds((start + j) * CH, CH)],
                              dst_v.at[j], ssem).wait()
        pltpu.sync_copy(ones_v, deg_sh.at[dst_v.at[j]], add=True)
        return 0

    lax.fori_loop(0, nch, body, 0)
    plsc.subcore_barrier()

    pltpu.sync_copy(deg_sh.at[pl.ds(sid * DPT, DPT)],
                    degp_hbm.at[cid, pl.ds(sid * DPT, DPT)])


@functools.partial(
    pl.kernel,
    out_type=jax.ShapeDtypeStruct((NC, NPAD, D), jnp.float32),
    mesh=_mesh,
    scratch_types=[
        pltpu.VMEM_SHARED((NPAD, D), jnp.float32),
        pltpu.VMEM((2, CH), jnp.int32),
        pltpu.VMEM((3, CH), jnp.int32),
        pltpu.VMEM((2, CH, D), jnp.float32),
        pltpu.SemaphoreType.DMA((2,)),
        pltpu.SemaphoreType.DMA((2,)),
        pltpu.SemaphoreType.DMA((3,)),
        pltpu.SemaphoreType.DMA((2,)),
    ],
)
def _scatter_kernel(hs_hbm, ei_hbm, p_hbm,
                    agg_sh, sidx_v, didx_v, rows_v, gsem, isem, dsem, asem):
    cid = lax.axis_index("c")
    sid = lax.axis_index("s")
    wid = cid * NS + sid
    start, nch = _tile_range(wid)

    def srcs(j):
        return ei_hbm.at[0, pl.ds((start + j) * CH, CH)]

    def dsts(j):
        return ei_hbm.at[1, pl.ds((start + j) * CH, CH)]

    pltpu.sync_copy(srcs(0), sidx_v.at[0])
    pltpu.sync_copy(dsts(0), didx_v.at[0])
    pltpu.async_copy(hs_hbm.at[sidx_v.at[0]], rows_v.at[0], gsem.at[0])
    pltpu.async_copy(srcs(1), sidx_v.at[1], isem.at[1])
    pltpu.async_copy(dsts(1), didx_v.at[1], dsem.at[1])

    z16 = jnp.zeros((16,), jnp.float32)

    def zbody(r, _):
        for c in range(D // 16):
            rows_v[1, r, pl.ds(c * 16, 16)] = z16
        return 0

    lax.fori_loop(0, CH, zbody, 0)
    for k in range(4):
        pltpu.sync_copy(rows_v.at[1],
                        agg_sh.at[pl.ds(sid * RPT + k * CH, CH)])
    pltpu.sync_copy(rows_v.at[1, pl.ds(0, RPT - 4 * CH)],
                    agg_sh.at[pl.ds(sid * RPT + 4 * CH, RPT - 4 * CH)])

    plsc.subcore_barrier()

    def body(j, _):
        b = lax.rem(j, 2)
        nb = lax.rem(j + 1, 2)
        m3 = lax.rem(j, 3)
        p3 = lax.rem(j + 2, 3)

        @pl.when(j >= 1)
        def _():
            pltpu.make_async_copy(
                rows_v.at[nb], agg_sh.at[didx_v.at[lax.rem(j - 1, 3)]],
                asem.at[nb]).wait()

        @pl.when(j + 1 < nch)
        def _():
            pltpu.make_async_copy(srcs(j + 1), sidx_v.at[nb],
                                  isem.at[nb]).wait()
            pltpu.async_copy(hs_hbm.at[sidx_v.at[nb]], rows_v.at[nb],
                             gsem.at[nb])

        pltpu.make_async_copy(hs_hbm.at[sidx_v.at[b]], rows_v.at[b],
                              gsem.at[b]).wait()

        @pl.when(j + 2 < nch)
        def _():
            pltpu.async_copy(srcs(j + 2), sidx_v.at[b], isem.at[b])

        @pl.when(j >= 1)
        def _():
            pltpu.make_async_copy(dsts(j), didx_v.at[m3], dsem.at[m3]).wait()

        pltpu.async_copy(rows_v.at[b], agg_sh.at[didx_v.at[m3]], asem.at[b],
                         add=True)

        @pl.when(j + 2 < nch)
        def _():
            pltpu.async_copy(dsts(j + 2), didx_v.at[p3], dsem.at[p3])

        return 0

    lax.fori_loop(0, nch, body, 0)

    lb = lax.rem(nch - 1, 2)
    lm = lax.rem(nch - 1, 3)
    pltpu.make_async_copy(rows_v.at[lb], agg_sh.at[didx_v.at[lm]],
                          asem.at[lb]).wait()
    plsc.subcore_barrier()

    pltpu.sync_copy(agg_sh.at[pl.ds(sid * RPT, RPT)],
                    p_hbm.at[cid, pl.ds(sid * RPT, RPT)])


def _mm_body(data_ref, wpre_ref, wpost_ref, h_ref):
    wc = jnp.dot(wpre_ref[...], wpost_ref[...], preferred_element_type=jnp.float32)
    h_ref[...] = jnp.dot(data_ref[...], wc, preferred_element_type=jnp.float32)


_mm_call = pl.pallas_call(
    _mm_body,
    out_shape=jax.ShapeDtypeStruct((N_NODES, D), jnp.float32),
)


def _isd_col(degp, n):
    deg = degp[0] + degp[1]
    isd = lax.rsqrt(jnp.maximum(deg, 1.0))
    return isd[:n, None]


def _scale_body(h_ref, degp_ref, hs_ref):
    hs_ref[0:N_NODES, :] = h_ref[...] * _isd_col(degp_ref[...], N_NODES)
    hs_ref[N_NODES:NPAD, :] = jnp.zeros((NPAD - N_NODES, D), jnp.float32)


_scale_call = pl.pallas_call(
    _scale_body,
    out_shape=jax.ShapeDtypeStruct((NPAD, D), jnp.float32),
)


def _post_body(p_ref, degp_ref, out_ref):
    s = p_ref[0] + p_ref[1]
    out_ref[...] = s[:N_NODES] * _isd_col(degp_ref[...], N_NODES)


_post_call = pl.pallas_call(
    _post_body,
    out_shape=jax.ShapeDtypeStruct((N_NODES, D), jnp.float32),
)


@jax.jit
def kernel(data, edge_index, W_pre, W_post):
    degp = _deg_kernel(edge_index)
    h = _mm_call(data, W_pre, W_post)
    hs = _scale_call(h, degp)
    p = _scatter_kernel(hs, edge_index)
    return _post_call(p, degp)


# speedup vs baseline: 2.2038x; 1.0307x over previous
import functools

import jax
import jax.numpy as jnp
from jax import lax
from jax.experimental import pallas as pl
from jax.experimental.pallas import tpu as pltpu
from jax.experimental.pallas import tpu_sc as plsc

N_NODES = 10000
N_EDGES = 320000
D = 128

NC = 2
NS = 16
NW = NC * NS
CH = 128
NCHUNK = N_EDGES // CH
FBASE = NCHUNK // NW
FEXTRA = NCHUNK - FBASE * NW

NPAD = 10112
RPT = NPAD // NS
NDEG = 10240
DPT = NDEG // NS

_mesh = plsc.VectorSubcoreMesh(core_axis_name="c", subcore_axis_name="s")


def _tile_range(wid):
    start = wid * FBASE + jnp.minimum(wid, FEXTRA)
    n = jnp.where(wid < FEXTRA, FBASE + 1, FBASE)
    return start, n


@functools.partial(
    pl.kernel,
    out_type=jax.ShapeDtypeStruct((NC, NDEG), jnp.float32),
    mesh=_mesh,
    scratch_types=[
        pltpu.VMEM_SHARED((NDEG,), jnp.float32),
        pltpu.VMEM((FBASE + 1, CH), jnp.int32),
        pltpu.VMEM((CH,), jnp.float32),
        pltpu.VMEM((DPT,), jnp.float32),
        pltpu.SemaphoreType.DMA,
        pltpu.SemaphoreType.DMA,
    ],
)
def _deg_kernel(ei_hbm, degp_hbm, deg_sh, dst_v, ones_v, zb_v, ssem, asem):
    cid = lax.axis_index("c")
    sid = lax.axis_index("s")
    wid = cid * NS + sid
    start, nch = _tile_range(wid)

    def sbody(j, _):
        pltpu.async_copy(ei_hbm.at[1, pl.ds((start + j) * CH, CH)],
                         dst_v.at[j], ssem)
        return 0

    lax.fori_loop(0, nch, sbody, 0)

    one16 = jnp.ones((16,), jnp.float32)
    z16 = jnp.zeros((16,), jnp.float32)
    for c in range(CH // 16):
        ones_v[pl.ds(c * 16, 16)] = one16

    def zvbody(c, _):
        zb_v[pl.ds(c * 16, 16)] = z16
        return 0

    lax.fori_loop(0, DPT // 16, zvbody, 0)
    pltpu.sync_copy(zb_v, deg_sh.at[pl.ds(sid * DPT, DPT)])
    plsc.subcore_barrier()

    def body(j, _):
        pltpu.make_async_copy(ei_hbm.at[1, pl.ds((start + j) * CH, CH)],
                              dst_v.at[j], ssem).wait()
        pltpu.async_copy(ones_v, deg_sh.at[dst_v.at[j]], asem, add=True)
        return 0

    lax.fori_loop(0, nch, body, 0)

    def dbody(j, _):
        pltpu.make_async_copy(ones_v, deg_sh.at[dst_v.at[j]], asem).wait()
        return 0

    lax.fori_loop(0, nch, dbody, 0)
    plsc.subcore_barrier()

    pltpu.sync_copy(deg_sh.at[pl.ds(sid * DPT, DPT)],
                    degp_hbm.at[cid, pl.ds(sid * DPT, DPT)])


@functools.partial(
    pl.kernel,
    out_type=jax.ShapeDtypeStruct((NC, NPAD, D), jnp.float32),
    mesh=_mesh,
    scratch_types=[
        pltpu.VMEM_SHARED((NPAD, D), jnp.float32),
        pltpu.VMEM((2, CH), jnp.int32),
        pltpu.VMEM((3, CH), jnp.int32),
        pltpu.VMEM((2, CH, D), jnp.float32),
        pltpu.SemaphoreType.DMA((2,)),
        pltpu.SemaphoreType.DMA((2,)),
        pltpu.SemaphoreType.DMA((3,)),
        pltpu.SemaphoreType.DMA((2,)),
    ],
)
def _scatter_kernel(hs_hbm, ei_hbm, p_hbm,
                    agg_sh, sidx_v, didx_v, rows_v, gsem, isem, dsem, asem):
    cid = lax.axis_index("c")
    sid = lax.axis_index("s")
    wid = cid * NS + sid
    start, nch = _tile_range(wid)

    def srcs(j):
        return ei_hbm.at[0, pl.ds((start + j) * CH, CH)]

    def dsts(j):
        return ei_hbm.at[1, pl.ds((start + j) * CH, CH)]

    pltpu.sync_copy(srcs(0), sidx_v.at[0])
    pltpu.sync_copy(dsts(0), didx_v.at[0])
    pltpu.async_copy(hs_hbm.at[sidx_v.at[0]], rows_v.at[0], gsem.at[0])
    pltpu.async_copy(srcs(1), sidx_v.at[1], isem.at[1])
    pltpu.async_copy(dsts(1), didx_v.at[1], dsem.at[1])

    z16 = jnp.zeros((16,), jnp.float32)

    def zbody(r, _):
        for c in range(D // 16):
            rows_v[1, r, pl.ds(c * 16, 16)] = z16
        return 0

    lax.fori_loop(0, CH, zbody, 0)
    for k in range(4):
        pltpu.sync_copy(rows_v.at[1],
                        agg_sh.at[pl.ds(sid * RPT + k * CH, CH)])
    pltpu.sync_copy(rows_v.at[1, pl.ds(0, RPT - 4 * CH)],
                    agg_sh.at[pl.ds(sid * RPT + 4 * CH, RPT - 4 * CH)])

    plsc.subcore_barrier()

    def body(j, _):
        b = lax.rem(j, 2)
        nb = lax.rem(j + 1, 2)
        m3 = lax.rem(j, 3)
        p3 = lax.rem(j + 2, 3)

        @pl.when(j >= 1)
        def _():
            pltpu.make_async_copy(
                rows_v.at[nb], agg_sh.at[didx_v.at[lax.rem(j - 1, 3)]],
                asem.at[nb]).wait()

        @pl.when(j + 1 < nch)
        def _():
            pltpu.make_async_copy(srcs(j + 1), sidx_v.at[nb],
                                  isem.at[nb]).wait()
            pltpu.async_copy(hs_hbm.at[sidx_v.at[nb]], rows_v.at[nb],
                             gsem.at[nb])

        pltpu.make_async_copy(hs_hbm.at[sidx_v.at[b]], rows_v.at[b],
                              gsem.at[b]).wait()

        @pl.when(j + 2 < nch)
        def _():
            pltpu.async_copy(srcs(j + 2), sidx_v.at[b], isem.at[b])

        @pl.when(j >= 1)
        def _():
            pltpu.make_async_copy(dsts(j), didx_v.at[m3], dsem.at[m3]).wait()

        pltpu.async_copy(rows_v.at[b], agg_sh.at[didx_v.at[m3]], asem.at[b],
                         add=True)

        @pl.when(j + 2 < nch)
        def _():
            pltpu.async_copy(dsts(j + 2), didx_v.at[p3], dsem.at[p3])

        return 0

    lax.fori_loop(0, nch, body, 0)

    lb = lax.rem(nch - 1, 2)
    lm = lax.rem(nch - 1, 3)
    pltpu.make_async_copy(rows_v.at[lb], agg_sh.at[didx_v.at[lm]],
                          asem.at[lb]).wait()
    plsc.subcore_barrier()

    pltpu.sync_copy(agg_sh.at[pl.ds(sid * RPT, RPT)],
                    p_hbm.at[cid, pl.ds(sid * RPT, RPT)])


def _mm_body(data_ref, wpre_ref, wpost_ref, h_ref):
    wc = jnp.dot(wpre_ref[...], wpost_ref[...], preferred_element_type=jnp.float32)
    h_ref[...] = jnp.dot(data_ref[...], wc, preferred_element_type=jnp.float32)


_mm_call = pl.pallas_call(
    _mm_body,
    out_shape=jax.ShapeDtypeStruct((N_NODES, D), jnp.float32),
)


def _isd_col(degp, n):
    deg = degp[0] + degp[1]
    isd = lax.rsqrt(jnp.maximum(deg, 1.0))
    return isd[:n, None]


def _scale_body(h_ref, degp_ref, hs_ref):
    hs_ref[0:N_NODES, :] = h_ref[...] * _isd_col(degp_ref[...], N_NODES)
    hs_ref[N_NODES:NPAD, :] = jnp.zeros((NPAD - N_NODES, D), jnp.float32)


_scale_call = pl.pallas_call(
    _scale_body,
    out_shape=jax.ShapeDtypeStruct((NPAD, D), jnp.float32),
)


def _post_body(p_ref, degp_ref, out_ref):
    s = p_ref[0] + p_ref[1]
    out_ref[...] = s[:N_NODES] * _isd_col(degp_ref[...], N_NODES)


_post_call = pl.pallas_call(
    _post_body,
    out_shape=jax.ShapeDtypeStruct((N_NODES, D), jnp.float32),
)


@jax.jit
def kernel(data, edge_index, W_pre, W_post):
    degp = _deg_kernel(edge_index)
    h = _mm_call(data, W_pre, W_post)
    hs = _scale_call(h, degp)
    p = _scatter_kernel(hs, edge_index)
    return _post_call(p, degp)
